# baseline (speedup 1.0000x reference)
import functools

import jax
import jax.numpy as jnp
from jax import lax
from jax.experimental import pallas as pl
from jax.experimental.pallas import tpu as pltpu
from jax.experimental.pallas import tpu_sc as plsc

KNN_K = 32


def _knn_body(xr_ref, yr_ref, xc_ref, yc_ref, idx_ref, *, n, k):
    xr = xr_ref[0]
    yr = yr_ref[0]
    xc = xc_ref[0]
    yc = yc_ref[0]
    bxr = xr.astype(jnp.bfloat16).astype(jnp.float32)
    byr = yr.astype(jnp.bfloat16).astype(jnp.float32)
    bxc = xc.astype(jnp.bfloat16).astype(jnp.float32)
    byc = yc.astype(jnp.bfloat16).astype(jnp.float32)
    inner2 = 2.0 * (bxc * bxr + byc * byr)
    xxr = xr * xr + yr * yr
    xxc = xc * xc + yc * yc
    neg = inner2 - xxc - xxr
    iota = lax.broadcasted_iota(jnp.int32, (n, n), 1)
    cols = []
    for _ in range(k):
        m = jnp.max(neg, axis=1, keepdims=True)
        am = jnp.min(jnp.where(neg == m, iota, n), axis=1,
                     keepdims=True)
        cols.append(am)
        neg = jnp.where(iota == am, -jnp.inf, neg)
    idx_ref[0] = jnp.concatenate(cols, axis=1)


def _transpose_body(x_ref, out_ref):
    out_ref[0] = jnp.transpose(x_ref[0], (1, 0))


def _gidx_body(m_ref, out_ref, *, n):
    bi = pl.program_id(0)
    out_ref[0] = jnp.transpose(m_ref[0], (1, 0)) + bi * n


def _sc_gather_sum(table, gidx, k, f):
    rows = gidx.shape[0]
    info = plsc.get_sparse_core_info()
    nc, ns = info.num_cores, info.num_subcores
    nw = nc * ns
    ch = rows // nw
    mesh = plsc.VectorSubcoreMesh(core_axis_name="c", subcore_axis_name="s")

    @functools.partial(
        pl.kernel,
        mesh=mesh,
        out_type=jax.ShapeDtypeStruct((rows, f), jnp.float32),
        scratch_types=[
            pltpu.VMEM((ch, k), jnp.int32),
            pltpu.VMEM((k, f), jnp.float32),
            pltpu.VMEM((1, f), jnp.float32),
            pltpu.SemaphoreType.DMA,
        ],
    )
    def gk(table_hbm, gidx_hbm, out_hbm, idx_v, buf, acc, sem):
        wid = lax.axis_index("s") * nc + lax.axis_index("c")
        base = wid * ch
        pltpu.sync_copy(gidx_hbm.at[pl.ds(base, ch)], idx_v)

        def row_body(r, carry):
            pltpu.async_copy(table_hbm.at[idx_v.at[r]], buf, sem).wait()

            def chunk_body(c, cc):
                off = c * 16
                s = buf[0, pl.ds(off, 16)]
                for j in range(1, k):
                    s = s + buf[j, pl.ds(off, 16)]
                acc[0, pl.ds(off, 16)] = s
                return cc

            lax.fori_loop(0, f // 16, chunk_body, 0)
            pltpu.sync_copy(acc, out_hbm.at[pl.ds(base + r, 1)])
            return carry

        lax.fori_loop(0, ch, row_body, 0)

    return gk(table, gidx)


def _trans_body(g_ref, feat_ref, wt_ref, b_ref, trans_ref, s1_ref, s2_ref,
                *, k):
    lap = feat_ref[0] - g_ref[0] * (1.0 / k)
    trans = jnp.dot(lap, wt_ref[...], preferred_element_type=jnp.float32)
    trans = trans + b_ref[...]
    trans_ref[0] = trans
    s1_ref[0] = jnp.sum(trans, axis=1, keepdims=True)
    s2_ref[0] = jnp.sum(trans * trans, axis=1, keepdims=True)


def _final_body(trans_ref, feat_ref, s1_ref, s2_ref, g_ref, be_ref, out_ref,
                *, nb, f):
    b = s1_ref.shape[0]
    s1 = s1_ref[0]
    s2 = s2_ref[0]
    for bi in range(1, b):
        s1 = s1 + s1_ref[bi]
        s2 = s2 + s2_ref[bi]
    denom = 1.0 / (b * f)
    mean = s1 * denom
    var = s2 * denom - mean * mean
    rstd = lax.rsqrt(var + 1e-5)
    trans = trans_ref[0]
    t = (trans - mean) * rstd * g_ref[0] + be_ref[0]
    out_ref[0] = feat_ref[0] + jnp.maximum(t, 0.0)


def kernel(xyz, feat, W, b, gamma, beta):
    B, N, _ = xyz.shape
    F = feat.shape[2]
    K = KNN_K
    RB = min(256, N)
    NB = N // RB

    xr = xyz[:, :, 0].reshape(B, 1, N)
    yr = xyz[:, :, 1].reshape(B, 1, N)
    xc = xyz[:, :, 0].reshape(B, N, 1)
    yc = xyz[:, :, 1].reshape(B, N, 1)

    idx = pl.pallas_call(
        functools.partial(_knn_body, n=N, k=K),
        grid=(B,),
        in_specs=[
            pl.BlockSpec((1, 1, N), lambda bi: (bi, 0, 0)),
            pl.BlockSpec((1, 1, N), lambda bi: (bi, 0, 0)),
            pl.BlockSpec((1, N, 1), lambda bi: (bi, 0, 0)),
            pl.BlockSpec((1, N, 1), lambda bi: (bi, 0, 0)),
        ],
        out_specs=pl.BlockSpec((1, N, K), lambda bi: (bi, 0, 0)),
        out_shape=jax.ShapeDtypeStruct((B, N, K), jnp.int32),
    )(xr, yr, xc, yc)

    m3 = idx.reshape(B, K, N)
    gidx = pl.pallas_call(
        functools.partial(_gidx_body, n=N),
        grid=(B,),
        in_specs=[pl.BlockSpec((1, K, N), lambda bi: (bi, 0, 0))],
        out_specs=pl.BlockSpec((1, N, K), lambda bi: (bi, 0, 0)),
        out_shape=jax.ShapeDtypeStruct((B, N, K), jnp.int32),
    )(m3).reshape(B * N, K)

    TB = min(256, N)
    featT = pl.pallas_call(
        _transpose_body,
        grid=(B, N // TB, F // TB),
        in_specs=[pl.BlockSpec((1, TB, TB), lambda bi, i, j: (bi, i, j))],
        out_specs=pl.BlockSpec((1, TB, TB), lambda bi, i, j: (bi, j, i)),
        out_shape=jax.ShapeDtypeStruct((B, F, N), jnp.float32),
    )(feat)

    wt = W.T
    b2 = b.reshape(1, F)

    g = _sc_gather_sum(featT.reshape(B * N, F), gidx, K, F).reshape(B, N, F)

    trans, s1, s2 = pl.pallas_call(
        functools.partial(_trans_body, k=K),
        grid=(B, NB),
        in_specs=[
            pl.BlockSpec((1, RB, F), lambda bi, i: (bi, i, 0)),
            pl.BlockSpec((1, RB, F), lambda bi, i: (bi, i, 0)),
            pl.BlockSpec((F, F), lambda bi, i: (0, 0)),
            pl.BlockSpec((1, F), lambda bi, i: (0, 0)),
        ],
        out_specs=[
            pl.BlockSpec((1, RB, F), lambda bi, i: (bi, i, 0)),
            pl.BlockSpec((1, RB, 1), lambda bi, i: (bi, i, 0)),
            pl.BlockSpec((1, RB, 1), lambda bi, i: (bi, i, 0)),
        ],
        out_shape=[
            jax.ShapeDtypeStruct((B, N, F), jnp.float32),
            jax.ShapeDtypeStruct((B, N, 1), jnp.float32),
            jax.ShapeDtypeStruct((B, N, 1), jnp.float32),
        ],
    )(g, feat, wt, b2)

    g3 = gamma.reshape(1, N, 1)
    be3 = beta.reshape(1, N, 1)

    out = pl.pallas_call(
        functools.partial(_final_body, nb=NB, f=F),
        grid=(B, NB),
        in_specs=[
            pl.BlockSpec((1, RB, F), lambda bi, i: (bi, i, 0)),
            pl.BlockSpec((1, RB, F), lambda bi, i: (bi, i, 0)),
            pl.BlockSpec((B, RB, 1), lambda bi, i: (0, i, 0)),
            pl.BlockSpec((B, RB, 1), lambda bi, i: (0, i, 0)),
            pl.BlockSpec((1, RB, 1), lambda bi, i: (0, i, 0)),
            pl.BlockSpec((1, RB, 1), lambda bi, i: (0, i, 0)),
        ],
        out_specs=pl.BlockSpec((1, RB, F), lambda bi, i: (bi, i, 0)),
        out_shape=jax.ShapeDtypeStruct((B, N, F), jnp.float32),
    )(trans, feat, s1, s2, g3, be3)
    return out


# speedup vs baseline: 1.2725x; 1.2725x over previous
import functools

import jax
import jax.numpy as jnp
from jax import lax
from jax.experimental import pallas as pl
from jax.experimental.pallas import tpu as pltpu
from jax.experimental.pallas import tpu_sc as plsc

KNN_K = 32


def _knn_body(xr_ref, yr_ref, xc_ref, yc_ref, idx_ref, *, n, k):
    xr = xr_ref[0]
    yr = yr_ref[0]
    xc = xc_ref[0]
    yc = yc_ref[0]
    bxr = xr.astype(jnp.bfloat16).astype(jnp.float32)
    byr = yr.astype(jnp.bfloat16).astype(jnp.float32)
    bxc = xc.astype(jnp.bfloat16).astype(jnp.float32)
    byc = yc.astype(jnp.bfloat16).astype(jnp.float32)
    inner2 = 2.0 * (bxc * bxr + byc * byr)
    xxr = xr * xr + yr * yr
    xxc = xc * xc + yc * yc
    neg = inner2 - xxc - xxr
    iota = lax.broadcasted_iota(jnp.int32, (n, n), 1)
    cols = []
    for _ in range(k):
        m = jnp.max(neg, axis=1, keepdims=True)
        am = jnp.min(jnp.where(neg == m, iota, n), axis=1,
                     keepdims=True)
        cols.append(am)
        neg = jnp.where(iota == am, -jnp.inf, neg)
    idx_ref[0] = jnp.concatenate(cols, axis=1)


def _transpose_body(x_ref, out_ref):
    out_ref[0] = jnp.transpose(x_ref[0], (1, 0))


def _gidx_body(m_ref, out_ref, *, n):
    bi = pl.program_id(0)
    out_ref[0] = jnp.transpose(m_ref[0], (1, 0)) + bi * n


def _sc_gather_sum(table, gidx, k, f):
    rows = gidx.shape[0]
    info = plsc.get_sparse_core_info()
    nc, ns = info.num_cores, info.num_subcores
    nw = nc * ns
    ch = rows // nw
    mesh = plsc.VectorSubcoreMesh(core_axis_name="c", subcore_axis_name="s")

    kh = k // 2

    @functools.partial(
        pl.kernel,
        mesh=mesh,
        out_type=jax.ShapeDtypeStruct((rows, f), jnp.float32),
        scratch_types=[
            pltpu.VMEM((ch, k), jnp.int32),
            pltpu.VMEM((2, kh, f), jnp.float32),
            pltpu.VMEM((1, f), jnp.float32),
            pltpu.SemaphoreType.DMA,
            pltpu.SemaphoreType.DMA,
        ],
    )
    def gk(table_hbm, gidx_hbm, out_hbm, idx_v, buf, acc, sem0, sem1):
        wid = lax.axis_index("s") * nc + lax.axis_index("c")
        base = wid * ch
        pltpu.sync_copy(gidx_hbm.at[pl.ds(base, ch)], idx_v)

        def start(r, h, slot, sem):
            pltpu.make_async_copy(
                table_hbm.at[idx_v.at[r, pl.ds(h * kh, kh)]],
                buf.at[slot], sem).start()

        def drain(slot, sem):
            pltpu.make_async_copy(
                table_hbm.at[idx_v.at[0, pl.ds(0, kh)]],
                buf.at[slot], sem).wait()

        def accum(slot, first):
            def chunk_body(c, cc):
                off = c * 16
                s = buf[slot, 0, pl.ds(off, 16)]
                for j in range(1, kh):
                    s = s + buf[slot, j, pl.ds(off, 16)]
                if first:
                    acc[0, pl.ds(off, 16)] = s
                else:
                    acc[0, pl.ds(off, 16)] += s
                return cc

            lax.fori_loop(0, f // 16, chunk_body, 0)

        start(0, 0, 0, sem0)

        def row_body(r, carry):
            drain(0, sem0)
            start(r, 1, 1, sem1)
            accum(0, True)
            drain(1, sem1)

            @pl.when(r + 1 < ch)
            def _():
                start(r + 1, 0, 0, sem0)

            accum(1, False)
            pltpu.sync_copy(acc, out_hbm.at[pl.ds(base + r, 1)])
            return carry

        lax.fori_loop(0, ch, row_body, 0)

    return gk(table, gidx)


def _trans_body(g_ref, feat_ref, wt_ref, b_ref, trans_ref, s1_ref, s2_ref,
                *, k):
    lap = feat_ref[0] - g_ref[0] * (1.0 / k)
    trans = jnp.dot(lap, wt_ref[...], preferred_element_type=jnp.float32)
    trans = trans + b_ref[...]
    trans_ref[0] = trans
    s1_ref[0] = jnp.sum(trans, axis=1, keepdims=True)
    s2_ref[0] = jnp.sum(trans * trans, axis=1, keepdims=True)


def _final_body(trans_ref, feat_ref, s1_ref, s2_ref, g_ref, be_ref, out_ref,
                *, nb, f):
    b = s1_ref.shape[0]
    s1 = s1_ref[0]
    s2 = s2_ref[0]
    for bi in range(1, b):
        s1 = s1 + s1_ref[bi]
        s2 = s2 + s2_ref[bi]
    denom = 1.0 / (b * f)
    mean = s1 * denom
    var = s2 * denom - mean * mean
    rstd = lax.rsqrt(var + 1e-5)
    trans = trans_ref[0]
    t = (trans - mean) * rstd * g_ref[0] + be_ref[0]
    out_ref[0] = feat_ref[0] + jnp.maximum(t, 0.0)


def kernel(xyz, feat, W, b, gamma, beta):
    B, N, _ = xyz.shape
    F = feat.shape[2]
    K = KNN_K
    RB = min(256, N)
    NB = N // RB

    xr = xyz[:, :, 0].reshape(B, 1, N)
    yr = xyz[:, :, 1].reshape(B, 1, N)
    xc = xyz[:, :, 0].reshape(B, N, 1)
    yc = xyz[:, :, 1].reshape(B, N, 1)

    idx = pl.pallas_call(
        functools.partial(_knn_body, n=N, k=K),
        grid=(B,),
        in_specs=[
            pl.BlockSpec((1, 1, N), lambda bi: (bi, 0, 0)),
            pl.BlockSpec((1, 1, N), lambda bi: (bi, 0, 0)),
            pl.BlockSpec((1, N, 1), lambda bi: (bi, 0, 0)),
            pl.BlockSpec((1, N, 1), lambda bi: (bi, 0, 0)),
        ],
        out_specs=pl.BlockSpec((1, N, K), lambda bi: (bi, 0, 0)),
        out_shape=jax.ShapeDtypeStruct((B, N, K), jnp.int32),
    )(xr, yr, xc, yc)

    m3 = idx.reshape(B, K, N)
    gidx = pl.pallas_call(
        functools.partial(_gidx_body, n=N),
        grid=(B,),
        in_specs=[pl.BlockSpec((1, K, N), lambda bi: (bi, 0, 0))],
        out_specs=pl.BlockSpec((1, N, K), lambda bi: (bi, 0, 0)),
        out_shape=jax.ShapeDtypeStruct((B, N, K), jnp.int32),
    )(m3).reshape(B * N, K)

    TB = min(256, N)
    featT = pl.pallas_call(
        _transpose_body,
        grid=(B, N // TB, F // TB),
        in_specs=[pl.BlockSpec((1, TB, TB), lambda bi, i, j: (bi, i, j))],
        out_specs=pl.BlockSpec((1, TB, TB), lambda bi, i, j: (bi, j, i)),
        out_shape=jax.ShapeDtypeStruct((B, F, N), jnp.float32),
    )(feat)

    wt = W.T
    b2 = b.reshape(1, F)

    g = _sc_gather_sum(featT.reshape(B * N, F), gidx, K, F).reshape(B, N, F)

    trans, s1, s2 = pl.pallas_call(
        functools.partial(_trans_body, k=K),
        grid=(B, NB),
        in_specs=[
            pl.BlockSpec((1, RB, F), lambda bi, i: (bi, i, 0)),
            pl.BlockSpec((1, RB, F), lambda bi, i: (bi, i, 0)),
            pl.BlockSpec((F, F), lambda bi, i: (0, 0)),
            pl.BlockSpec((1, F), lambda bi, i: (0, 0)),
        ],
        out_specs=[
            pl.BlockSpec((1, RB, F), lambda bi, i: (bi, i, 0)),
            pl.BlockSpec((1, RB, 1), lambda bi, i: (bi, i, 0)),
            pl.BlockSpec((1, RB, 1), lambda bi, i: (bi, i, 0)),
        ],
        out_shape=[
            jax.ShapeDtypeStruct((B, N, F), jnp.float32),
            jax.ShapeDtypeStruct((B, N, 1), jnp.float32),
            jax.ShapeDtypeStruct((B, N, 1), jnp.float32),
        ],
    )(g, feat, wt, b2)

    g3 = gamma.reshape(1, N, 1)
    be3 = beta.reshape(1, N, 1)

    out = pl.pallas_call(
        functools.partial(_final_body, nb=NB, f=F),
        grid=(B, NB),
        in_specs=[
            pl.BlockSpec((1, RB, F), lambda bi, i: (bi, i, 0)),
            pl.BlockSpec((1, RB, F), lambda bi, i: (bi, i, 0)),
            pl.BlockSpec((B, RB, 1), lambda bi, i: (0, i, 0)),
            pl.BlockSpec((B, RB, 1), lambda bi, i: (0, i, 0)),
            pl.BlockSpec((1, RB, 1), lambda bi, i: (0, i, 0)),
            pl.BlockSpec((1, RB, 1), lambda bi, i: (0, i, 0)),
        ],
        out_specs=pl.BlockSpec((1, RB, F), lambda bi, i: (bi, i, 0)),
        out_shape=jax.ShapeDtypeStruct((B, N, F), jnp.float32),
    )(trans, feat, s1, s2, g3, be3)
    return out
